# Initial kernel scaffold; baseline (speedup 1.0000x reference)
import jax
import jax.numpy as jnp
from jax.experimental import pallas as pl


def kernel(log_alpha):
    raise NotImplementedError("write your pallas kernel here")


# speedup vs baseline: 37.9372x; 37.9372x over previous
import functools
import math

import jax
import jax.numpy as jnp
from jax import lax
from jax.experimental import pallas as pl
from jax.experimental.pallas import tpu as pltpu
import jax.experimental.pallas.tpu_sc as plsc

N_IN = 1000000
NPAD = 1 << 20
NROWS = NPAD // 128
BETA = 2.0 / 3.0
BIAS = -BETA * math.log(0.1 / 1.1)

NW = 32
PER_W = NPAD // NW
CHUNK = 2048
NCHUNK = PER_W // CHUNK
HC = 16384
HR = 65536

BLK0 = 512
GRID0 = NROWS // BLK0


def _sigmoid(z):
    return 1.0 / (1.0 + jnp.exp(-z))


def _k0_body(x_ref, soft_ref, l0_ref, acc_ref):
    pid = pl.program_id(0)

    @pl.when(pid == 0)
    def _():
        acc_ref[0] = 0.0

    x = x_ref[...]
    s1 = _sigmoid(x + BIAS)
    r = lax.broadcasted_iota(jnp.int32, (BLK0, 128), 0)
    c = lax.broadcasted_iota(jnp.int32, (BLK0, 128), 1)
    flat = (pid * BLK0 + r) * 128 + c
    s1 = jnp.where(flat < N_IN, s1, 0.0)
    acc_ref[0] += jnp.sum(s1)
    soft_ref[...] = lax.bitcast_convert_type(_sigmoid((x / BETA) * 0.8),
                                             jnp.int32)

    @pl.when(pid == pl.num_programs(0) - 1)
    def _():
        l0_ref[0] = acc_ref[0]


def _k0(x2d):
    return pl.pallas_call(
        _k0_body,
        grid=(GRID0,),
        in_specs=[pl.BlockSpec((BLK0, 128), lambda i: (i, 0))],
        out_specs=[
            pl.BlockSpec((BLK0, 128), lambda i: (i, 0)),
            pl.BlockSpec(memory_space=pltpu.SMEM),
        ],
        out_shape=[
            jax.ShapeDtypeStruct((NROWS, 128), jnp.int32),
            jax.ShapeDtypeStruct((1,), jnp.float32),
        ],
        scratch_shapes=[pltpu.SMEM((1,), jnp.float32)],
    )(x2d)


_SC_MESH = plsc.VectorSubcoreMesh(core_axis_name="c", subcore_axis_name="s", num_cores=2, num_subcores=16)


@functools.partial(
    pl.kernel,
    out_type=jax.ShapeDtypeStruct((NW, HC), jnp.int32),
    mesh=_SC_MESH,
    compiler_params=pltpu.CompilerParams(needs_layout_passes=False),
    scratch_types=[
        pltpu.VMEM((CHUNK,), jnp.int32),
        pltpu.VMEM((HC,), jnp.int32),
    ],
)
def _k1(soft_hbm, hist_hbm, buf, hist):
    wid = lax.axis_index("s") * 2 + lax.axis_index("c")
    zeros16 = jnp.zeros((16,), jnp.int32)
    ones16 = jnp.ones((16,), jnp.int32)

    def zb(i, carry):
        for j in range(8):
            hist[pl.ds(i * 128 + j * 16, 16)] = zeros16
        return carry

    lax.fori_loop(0, HC // 128, zb, 0)

    base = wid * PER_W

    def cb(ci, carry):
        pltpu.sync_copy(soft_hbm.at[pl.ds(base + ci * CHUNK, CHUNK)], buf)

        def vb(vi, c2):
            u = buf[pl.ds(vi * 16, 16)]
            b = jnp.right_shift(u, 16)
            plsc.addupdate_scatter(hist, [b], ones16)
            return c2

        lax.fori_loop(0, CHUNK // 16, vb, 0)
        return carry

    lax.fori_loop(0, NCHUNK, cb, 0)
    pltpu.sync_copy(hist, hist_hbm.at[wid])


def _triangulars(n):
    r = lax.broadcasted_iota(jnp.int32, (n, n), 0)
    c = lax.broadcasted_iota(jnp.int32, (n, n), 1)
    upper = (r < c).astype(jnp.float32)
    lower = (c < r).astype(jnp.float32)
    return upper, lower


def _k2_body(h_ref, l0_ref, out_ref, bvec_ref):
    h = jnp.sum(h_ref[...].astype(jnp.float32), axis=0)
    up, lo = _triangulars(128)
    inrow = jnp.dot(h, up, preferred_element_type=jnp.float32,
                    precision=lax.Precision.HIGHEST)
    rowsum = jnp.sum(h, axis=1, keepdims=True)
    rowoffs = jnp.dot(lo, rowsum, preferred_element_type=jnp.float32,
                    precision=lax.Precision.HIGHEST)
    cum_excl = rowoffs + inrow
    cum_incl = cum_excl + h

    kf = jnp.clip(jnp.round(jnp.float32(N_IN) - l0_ref[0]), 0.0,
                  jnp.float32(N_IN))
    ki = kf.astype(jnp.int32)

    r = lax.broadcasted_iota(jnp.int32, (128, 128), 0)
    c = lax.broadcasted_iota(jnp.int32, (128, 128), 1)
    flat = r * 128 + c
    sel = cum_incl >= kf
    bkt = jnp.min(jnp.where(sel, flat, jnp.int32(1 << 30)))
    cl = jnp.sum(jnp.where(flat == bkt, cum_excl, 0.0)).astype(jnp.int32)

    out_ref[0] = ki
    out_ref[1] = bkt
    out_ref[2] = cl
    for i in range(16):
        bvec_ref[i] = bkt


def _k2(hists3d, l0):
    return pl.pallas_call(
        _k2_body,
        in_specs=[
            pl.BlockSpec(memory_space=pltpu.VMEM),
            pl.BlockSpec(memory_space=pltpu.SMEM),
        ],
        out_specs=[
            pl.BlockSpec(memory_space=pltpu.SMEM),
            pl.BlockSpec(memory_space=pltpu.SMEM),
        ],
        out_shape=[
            jax.ShapeDtypeStruct((8,), jnp.int32),
            jax.ShapeDtypeStruct((16,), jnp.int32),
        ],
    )(hists3d, l0)


@functools.partial(
    pl.kernel,
    out_type=jax.ShapeDtypeStruct((NW, HR), jnp.int32),
    mesh=_SC_MESH,
    compiler_params=pltpu.CompilerParams(needs_layout_passes=False),
    scratch_types=[
        pltpu.VMEM((CHUNK,), jnp.int32),
        pltpu.VMEM((HR,), jnp.int32),
        pltpu.VMEM((16,), jnp.int32),
    ],
)
def _k3(soft_hbm, bvec_hbm, rhist_hbm, buf, hist, bv):
    wid = lax.axis_index("s") * 2 + lax.axis_index("c")
    zeros16 = jnp.zeros((16,), jnp.int32)
    ones16 = jnp.ones((16,), jnp.int32)

    def zb(i, carry):
        for j in range(8):
            hist[pl.ds(i * 128 + j * 16, 16)] = zeros16
        return carry

    lax.fori_loop(0, HR // 128, zb, 0)

    pltpu.sync_copy(bvec_hbm, bv)
    b16 = bv[...]
    base = wid * PER_W

    def cb(ci, carry):
        pltpu.sync_copy(soft_hbm.at[pl.ds(base + ci * CHUNK, CHUNK)], buf)

        def vb(vi, c2):
            u = buf[pl.ds(vi * 16, 16)]
            hi = jnp.right_shift(u, 16)
            lov = jnp.bitwise_and(u, 65535)
            m = hi == b16
            plsc.addupdate_scatter(hist, [lov], ones16, mask=m)
            return c2

        lax.fori_loop(0, CHUNK // 16, vb, 0)
        return carry

    lax.fori_loop(0, NCHUNK, cb, 0)
    pltpu.sync_copy(hist, rhist_hbm.at[wid])


def _k4_body(h_ref, sc_ref, out_ref):
    h = jnp.sum(h_ref[...].astype(jnp.float32), axis=0)
    up128, _ = _triangulars(128)
    _, lo512 = _triangulars(512)
    inrow = jnp.dot(h, up128, preferred_element_type=jnp.float32,
                    precision=lax.Precision.HIGHEST)
    rowsum = jnp.sum(h, axis=1, keepdims=True)
    rowoffs = jnp.dot(lo512, rowsum, preferred_element_type=jnp.float32,
                    precision=lax.Precision.HIGHEST)
    cum_excl = rowoffs + inrow
    cum_incl = cum_excl + h

    k = sc_ref[0]
    bkt = sc_ref[1]
    cl = sc_ref[2]
    kpf = (k - cl).astype(jnp.float32)

    r = lax.broadcasted_iota(jnp.int32, (512, 128), 0)
    c = lax.broadcasted_iota(jnp.int32, (512, 128), 1)
    flat = r * 128 + c
    sel = cum_incl >= kpf
    tlow = jnp.min(jnp.where(sel, flat, jnp.int32(1 << 30)))
    excl_at = jnp.sum(jnp.where(flat == tlow, cum_excl, 0.0)).astype(jnp.int32)

    out_ref[0] = bkt * 65536 + tlow
    out_ref[1] = k - (cl + excl_at)


def _k4(rhists3d, scalars):
    return pl.pallas_call(
        _k4_body,
        in_specs=[
            pl.BlockSpec(memory_space=pltpu.VMEM),
            pl.BlockSpec(memory_space=pltpu.SMEM),
        ],
        out_specs=pl.BlockSpec(memory_space=pltpu.SMEM),
        out_shape=jax.ShapeDtypeStruct((8,), jnp.int32),
    )(rhists3d, scalars)


def _k5_body(soft_ref, sc_ref, out_ref, carry_ref):
    pid = pl.program_id(0)

    @pl.when(pid == 0)
    def _():
        carry_ref[0] = 0.0

    thr = sc_ref[0]
    rr = sc_ref[1]
    u = soft_ref[...]
    s = lax.bitcast_convert_type(u, jnp.float32)
    eq = u == thr
    eqf = eq.astype(jnp.float32)
    up128, _ = _triangulars(128)
    _, lo512 = _triangulars(BLK0)
    inrow = jnp.dot(eqf, up128, preferred_element_type=jnp.float32,
                    precision=lax.Precision.HIGHEST)
    rowsum = jnp.sum(eqf, axis=1, keepdims=True)
    rowoffs = jnp.dot(lo512, rowsum, preferred_element_type=jnp.float32,
                    precision=lax.Precision.HIGHEST)
    pre = carry_ref[0] + rowoffs + inrow
    zero = (u < thr) | (eq & (pre < rr.astype(jnp.float32)))
    out_ref[...] = jnp.where(zero, 0.0, s)
    carry_ref[0] += jnp.sum(eqf)


def _k5(soft2d, scalars):
    return pl.pallas_call(
        _k5_body,
        grid=(GRID0,),
        in_specs=[
            pl.BlockSpec((BLK0, 128), lambda i: (i, 0)),
            pl.BlockSpec(memory_space=pltpu.SMEM),
        ],
        out_specs=pl.BlockSpec((BLK0, 128), lambda i: (i, 0)),
        out_shape=jax.ShapeDtypeStruct((NROWS, 128), jnp.float32),
        scratch_shapes=[pltpu.SMEM((1,), jnp.float32)],
    )(soft2d, scalars)


@jax.jit
def kernel(log_alpha):
    xpad = jnp.pad(log_alpha, (0, NPAD - N_IN), constant_values=200.0)
    x2d = xpad.reshape(NROWS, 128)
    soft2d, l0 = _k0(x2d)
    soft1d = soft2d.reshape(NPAD)
    hists = _k1(soft1d)
    scalars2, bvec = _k2(hists.reshape(NW, 128, 128), l0)
    rhists = _k3(soft1d, bvec)
    scalars4 = _k4(rhists.reshape(NW, 512, 128), scalars2)
    out2d = _k5(soft2d, scalars4)
    return out2d.reshape(NPAD)[:N_IN]


# speedup vs baseline: 38.2993x; 1.0095x over previous
import functools
import math

import jax
import jax.numpy as jnp
from jax import lax
from jax.experimental import pallas as pl
from jax.experimental.pallas import tpu as pltpu
import jax.experimental.pallas.tpu_sc as plsc

N_IN = 1000000
NPAD = 1 << 20
NROWS = NPAD // 128
BETA = 2.0 / 3.0
BIAS = -BETA * math.log(0.1 / 1.1)

NW = 32
PER_W = NPAD // NW
CHUNK = 2048
NCHUNK = PER_W // CHUNK
HC = 16384
HR = 65536

BLK0 = 512
GRID0 = NROWS // BLK0


def _sigmoid(z):
    return 1.0 / (1.0 + jnp.exp(-z))


def _k0_body(x_ref, soft_ref, l0_ref, acc_ref):
    pid = pl.program_id(0)

    @pl.when(pid == 0)
    def _():
        acc_ref[0] = 0.0

    x = x_ref[...]
    s1 = _sigmoid(x + BIAS)
    r = lax.broadcasted_iota(jnp.int32, (BLK0, 128), 0)
    c = lax.broadcasted_iota(jnp.int32, (BLK0, 128), 1)
    flat = (pid * BLK0 + r) * 128 + c
    s1 = jnp.where(flat < N_IN, s1, 0.0)
    acc_ref[0] += jnp.sum(s1)
    soft_ref[...] = lax.bitcast_convert_type(_sigmoid((x / BETA) * 0.8),
                                             jnp.int32)

    @pl.when(pid == pl.num_programs(0) - 1)
    def _():
        l0_ref[0] = acc_ref[0]


def _k0(x2d):
    return pl.pallas_call(
        _k0_body,
        grid=(GRID0,),
        in_specs=[pl.BlockSpec((BLK0, 128), lambda i: (i, 0))],
        out_specs=[
            pl.BlockSpec((BLK0, 128), lambda i: (i, 0)),
            pl.BlockSpec(memory_space=pltpu.SMEM),
        ],
        out_shape=[
            jax.ShapeDtypeStruct((NROWS, 128), jnp.int32),
            jax.ShapeDtypeStruct((1,), jnp.float32),
        ],
        scratch_shapes=[pltpu.SMEM((1,), jnp.float32)],
    )(x2d)


_SC_MESH = plsc.VectorSubcoreMesh(core_axis_name="c", subcore_axis_name="s", num_cores=2, num_subcores=16)


@functools.partial(
    pl.kernel,
    out_type=jax.ShapeDtypeStruct((NW, HC), jnp.int32),
    mesh=_SC_MESH,
    compiler_params=pltpu.CompilerParams(needs_layout_passes=False),
    scratch_types=[
        pltpu.VMEM((CHUNK,), jnp.int32),
        pltpu.VMEM((HC,), jnp.int32),
    ],
)
def _k1(soft_hbm, hist_hbm, buf, hist):
    wid = lax.axis_index("s") * 2 + lax.axis_index("c")
    zeros16 = jnp.zeros((16,), jnp.int32)
    ones16 = jnp.ones((16,), jnp.int32)

    def zb(i, carry):
        for j in range(8):
            hist[pl.ds(i * 128 + j * 16, 16)] = zeros16
        return carry

    lax.fori_loop(0, HC // 128, zb, 0)

    base = wid * PER_W

    def cb(ci, carry):
        pltpu.sync_copy(soft_hbm.at[pl.ds(base + ci * CHUNK, CHUNK)], buf)

        def vb(vi, c2):
            for j in range(8):
                u = buf[pl.ds(vi * 128 + j * 16, 16)]
                b = jnp.right_shift(u, 16)
                plsc.addupdate_scatter(hist, [b], ones16)
            return c2

        lax.fori_loop(0, CHUNK // 128, vb, 0)
        return carry

    lax.fori_loop(0, NCHUNK, cb, 0)
    pltpu.sync_copy(hist, hist_hbm.at[wid])


def _triangulars(n):
    r = lax.broadcasted_iota(jnp.int32, (n, n), 0)
    c = lax.broadcasted_iota(jnp.int32, (n, n), 1)
    upper = (r < c).astype(jnp.float32)
    lower = (c < r).astype(jnp.float32)
    return upper, lower


def _k2_body(h_ref, l0_ref, out_ref, bvec_ref):
    h = jnp.sum(h_ref[...].astype(jnp.float32), axis=0)
    up, lo = _triangulars(128)
    inrow = jnp.dot(h, up, preferred_element_type=jnp.float32,
                    precision=lax.Precision.HIGHEST)
    rowsum = jnp.sum(h, axis=1, keepdims=True)
    rowoffs = jnp.dot(lo, rowsum, preferred_element_type=jnp.float32,
                    precision=lax.Precision.HIGHEST)
    cum_excl = rowoffs + inrow
    cum_incl = cum_excl + h

    kf = jnp.clip(jnp.round(jnp.float32(N_IN) - l0_ref[0]), 0.0,
                  jnp.float32(N_IN))
    ki = kf.astype(jnp.int32)

    r = lax.broadcasted_iota(jnp.int32, (128, 128), 0)
    c = lax.broadcasted_iota(jnp.int32, (128, 128), 1)
    flat = r * 128 + c
    sel = cum_incl >= kf
    bkt = jnp.min(jnp.where(sel, flat, jnp.int32(1 << 30)))
    cl = jnp.sum(jnp.where(flat == bkt, cum_excl, 0.0)).astype(jnp.int32)

    out_ref[0] = ki
    out_ref[1] = bkt
    out_ref[2] = cl
    for i in range(16):
        bvec_ref[i] = bkt


def _k2(hists3d, l0):
    return pl.pallas_call(
        _k2_body,
        in_specs=[
            pl.BlockSpec(memory_space=pltpu.VMEM),
            pl.BlockSpec(memory_space=pltpu.SMEM),
        ],
        out_specs=[
            pl.BlockSpec(memory_space=pltpu.SMEM),
            pl.BlockSpec(memory_space=pltpu.SMEM),
        ],
        out_shape=[
            jax.ShapeDtypeStruct((8,), jnp.int32),
            jax.ShapeDtypeStruct((16,), jnp.int32),
        ],
    )(hists3d, l0)


@functools.partial(
    pl.kernel,
    out_type=jax.ShapeDtypeStruct((NW, HR), jnp.int32),
    mesh=_SC_MESH,
    compiler_params=pltpu.CompilerParams(needs_layout_passes=False),
    scratch_types=[
        pltpu.VMEM((CHUNK,), jnp.int32),
        pltpu.VMEM((HR,), jnp.int32),
        pltpu.VMEM((16,), jnp.int32),
    ],
)
def _k3(soft_hbm, bvec_hbm, rhist_hbm, buf, hist, bv):
    wid = lax.axis_index("s") * 2 + lax.axis_index("c")
    zeros16 = jnp.zeros((16,), jnp.int32)
    ones16 = jnp.ones((16,), jnp.int32)

    def zb(i, carry):
        for j in range(8):
            hist[pl.ds(i * 128 + j * 16, 16)] = zeros16
        return carry

    lax.fori_loop(0, HR // 128, zb, 0)

    pltpu.sync_copy(bvec_hbm, bv)
    b16 = bv[...]
    base = wid * PER_W

    def cb(ci, carry):
        pltpu.sync_copy(soft_hbm.at[pl.ds(base + ci * CHUNK, CHUNK)], buf)

        def vb(vi, c2):
            for j in range(8):
                u = buf[pl.ds(vi * 128 + j * 16, 16)]
                hi = jnp.right_shift(u, 16)
                lov = jnp.bitwise_and(u, 65535)
                m = hi == b16
                plsc.addupdate_scatter(hist, [lov], ones16, mask=m)
            return c2

        lax.fori_loop(0, CHUNK // 128, vb, 0)
        return carry

    lax.fori_loop(0, NCHUNK, cb, 0)
    pltpu.sync_copy(hist, rhist_hbm.at[wid])


def _k4_body(h_ref, sc_ref, out_ref):
    h = jnp.sum(h_ref[...].astype(jnp.float32), axis=0)
    up128, _ = _triangulars(128)
    _, lo512 = _triangulars(512)
    inrow = jnp.dot(h, up128, preferred_element_type=jnp.float32,
                    precision=lax.Precision.HIGHEST)
    rowsum = jnp.sum(h, axis=1, keepdims=True)
    rowoffs = jnp.dot(lo512, rowsum, preferred_element_type=jnp.float32,
                    precision=lax.Precision.HIGHEST)
    cum_excl = rowoffs + inrow
    cum_incl = cum_excl + h

    k = sc_ref[0]
    bkt = sc_ref[1]
    cl = sc_ref[2]
    kpf = (k - cl).astype(jnp.float32)

    r = lax.broadcasted_iota(jnp.int32, (512, 128), 0)
    c = lax.broadcasted_iota(jnp.int32, (512, 128), 1)
    flat = r * 128 + c
    sel = cum_incl >= kpf
    tlow = jnp.min(jnp.where(sel, flat, jnp.int32(1 << 30)))
    excl_at = jnp.sum(jnp.where(flat == tlow, cum_excl, 0.0)).astype(jnp.int32)

    out_ref[0] = bkt * 65536 + tlow
    out_ref[1] = k - (cl + excl_at)


def _k4(rhists3d, scalars):
    return pl.pallas_call(
        _k4_body,
        in_specs=[
            pl.BlockSpec(memory_space=pltpu.VMEM),
            pl.BlockSpec(memory_space=pltpu.SMEM),
        ],
        out_specs=pl.BlockSpec(memory_space=pltpu.SMEM),
        out_shape=jax.ShapeDtypeStruct((8,), jnp.int32),
    )(rhists3d, scalars)


def _k5_body(soft_ref, sc_ref, out_ref, carry_ref):
    pid = pl.program_id(0)

    @pl.when(pid == 0)
    def _():
        carry_ref[0] = 0.0

    thr = sc_ref[0]
    rr = sc_ref[1]
    u = soft_ref[...]
    s = lax.bitcast_convert_type(u, jnp.float32)
    eq = u == thr
    eqf = eq.astype(jnp.float32)
    up128, _ = _triangulars(128)
    _, lo512 = _triangulars(BLK0)
    inrow = jnp.dot(eqf, up128, preferred_element_type=jnp.float32,
                    precision=lax.Precision.HIGHEST)
    rowsum = jnp.sum(eqf, axis=1, keepdims=True)
    rowoffs = jnp.dot(lo512, rowsum, preferred_element_type=jnp.float32,
                    precision=lax.Precision.HIGHEST)
    pre = carry_ref[0] + rowoffs + inrow
    zero = (u < thr) | (eq & (pre < rr.astype(jnp.float32)))
    out_ref[...] = jnp.where(zero, 0.0, s)
    carry_ref[0] += jnp.sum(eqf)


def _k5(soft2d, scalars):
    return pl.pallas_call(
        _k5_body,
        grid=(GRID0,),
        in_specs=[
            pl.BlockSpec((BLK0, 128), lambda i: (i, 0)),
            pl.BlockSpec(memory_space=pltpu.SMEM),
        ],
        out_specs=pl.BlockSpec((BLK0, 128), lambda i: (i, 0)),
        out_shape=jax.ShapeDtypeStruct((NROWS, 128), jnp.float32),
        scratch_shapes=[pltpu.SMEM((1,), jnp.float32)],
    )(soft2d, scalars)


@jax.jit
def kernel(log_alpha):
    xpad = jnp.pad(log_alpha, (0, NPAD - N_IN), constant_values=200.0)
    x2d = xpad.reshape(NROWS, 128)
    soft2d, l0 = _k0(x2d)
    soft1d = soft2d.reshape(NPAD)
    hists = _k1(soft1d)
    scalars2, bvec = _k2(hists.reshape(NW, 128, 128), l0)
    rhists = _k3(soft1d, bvec)
    scalars4 = _k4(rhists.reshape(NW, 512, 128), scalars2)
    out2d = _k5(soft2d, scalars4)
    return out2d.reshape(NPAD)[:N_IN]
